# baseline (speedup 1.0000x reference)
import functools

import jax
import jax.numpy as jnp
from jax import lax
from jax.experimental import pallas as pl
from jax.experimental.pallas import tpu as pltpu
from jax.experimental.pallas import tpu_sc as plsc

NW = 32
CHUNK = 256
DPAD = 128


def _gather_kernel(B, V, D):
    b_per_w = B // NW
    n_chunks = b_per_w // CHUNK
    mesh = plsc.VectorSubcoreMesh(core_axis_name="c", subcore_axis_name="s")

    scratch = (
        [pltpu.VMEM((CHUNK,), jnp.int32) for _ in range(2)]
        + [pltpu.VMEM((CHUNK, DPAD), jnp.float32) for _ in range(2)]
        + [pltpu.VMEM((D, CHUNK), jnp.float32) for _ in range(2)]
        + [pltpu.SemaphoreType.DMA for _ in range(6)]
    )

    @functools.partial(
        pl.kernel,
        out_type=jax.ShapeDtypeStruct((D, B), jnp.float32),
        mesh=mesh,
        scratch_types=scratch,
        compiler_params=pltpu.CompilerParams(use_tc_tiling_on_sc=True, needs_layout_passes=False),
    )
    def k(idx_hbm, table_hbm, out_hbm, *refs):
        idx_bufs = refs[0:2]
        rows_bufs = refs[2:4]
        tr_bufs = refs[4:6]
        sem_i = refs[6:8]
        sem_g = refs[8:10]
        sem_o = refs[10:12]

        nc = 2
        wid = lax.axis_index("s") * nc + lax.axis_index("c")
        base = wid * b_per_w

        def issue_idx(g, b):
            pltpu.async_copy(
                idx_hbm.at[pl.ds(base + g * CHUNK, CHUNK)], idx_bufs[b], sem_i[b]
            )

        def wait_idx(b):
            pltpu.make_async_copy(
                idx_hbm.at[pl.ds(0, CHUNK)], idx_bufs[b], sem_i[b]
            ).wait()

        def issue_gather(b):
            pltpu.async_copy(table_hbm.at[idx_bufs[b]], rows_bufs[b], sem_g[b])

        def wait_gather(b):
            pltpu.make_async_copy(
                table_hbm.at[idx_bufs[b]], rows_bufs[b], sem_g[b]
            ).wait()

        def issue_out(g, b):
            pltpu.async_copy(
                tr_bufs[b],
                out_hbm.at[:, pl.ds(base + g * CHUNK, CHUNK)],
                sem_o[b],
            )

        def wait_out(b):
            pltpu.make_async_copy(
                tr_bufs[b], out_hbm.at[:, pl.ds(0, CHUNK)], sem_o[b]
            ).wait()

        base_vecs = [
            lax.iota(jnp.int32, 16) + c0 for c0 in range(0, CHUNK, 16)
        ]

        def transpose_chunk(rows_buf, tr_buf):
            @plsc.parallel_loop(0, D // 8, 1, unroll=2)
            def dbody(t):
                d0 = pl.multiple_of(t * 8, 8)
                for k in range(8):
                    d = d0 + k
                    col = jnp.full((16,), d, dtype=jnp.int32)
                    for gi, bv in enumerate(base_vecs):
                        v = plsc.load_gather(rows_buf, [bv, col])
                        tr_buf[d, pl.ds(gi * 16, 16)] = v

        issue_idx(0, 0)
        wait_idx(0)
        issue_gather(0)
        issue_idx(1, 1)

        def body(g, carry):
            b = lax.rem(g, 2)
            @pl.when(g + 1 < n_chunks)
            def _():
                for bb in range(2):
                    @pl.when(b != bb)
                    def _():
                        wait_idx(bb)
                        issue_gather(bb)

            @pl.when(g + 2 < n_chunks)
            def _():
                for bb in range(2):
                    @pl.when(b == bb)
                    def _():
                        wait_gather(bb)
                        issue_idx(g + 2, bb)

            for bb in range(2):
                @pl.when(b == bb)
                def _():
                    @pl.when(g + 2 >= n_chunks)
                    def _():
                        wait_gather(bb)
                    @pl.when(g >= 2)
                    def _():
                        wait_out(bb)
                    transpose_chunk(rows_bufs[bb], tr_bufs[bb])
                    issue_out(g, bb)
            return carry

        lax.fori_loop(0, n_chunks, body, 0)

        for bb in range(2):
            wait_out(bb)

    return k


def kernel(patch_indices, patch_embeddings):
    B = patch_indices.shape[0]
    V, D = patch_embeddings.shape
    idx = patch_indices.astype(jnp.int32)
    table_pad = jnp.pad(patch_embeddings, ((0, 0), (0, DPAD - D)))
    out_t = _gather_kernel(B, V, D)(idx, table_pad)
    return out_t.T


# speedup vs baseline: 1.0529x; 1.0529x over previous
import functools

import jax
import jax.numpy as jnp
from jax import lax
from jax.experimental import pallas as pl
from jax.experimental.pallas import tpu as pltpu
from jax.experimental.pallas import tpu_sc as plsc

NW = 32
CHUNK = 256
DPAD = 128


def _gather_kernel(B, V, D):
    b_per_w = B // NW
    n_chunks = b_per_w // CHUNK
    mesh = plsc.VectorSubcoreMesh(core_axis_name="c", subcore_axis_name="s")

    scratch = (
        [pltpu.VMEM((CHUNK,), jnp.int32) for _ in range(2)]
        + [pltpu.VMEM((CHUNK, DPAD), jnp.float32) for _ in range(2)]
        + [pltpu.VMEM((D, CHUNK), jnp.float32) for _ in range(2)]
        + [pltpu.VMEM((CHUNK * 64,), jnp.float32)]
        + [pltpu.SemaphoreType.DMA for _ in range(6)]
    )

    @functools.partial(
        pl.kernel,
        out_type=jax.ShapeDtypeStruct((D, B), jnp.float32),
        mesh=mesh,
        scratch_types=scratch,
        compiler_params=pltpu.CompilerParams(use_tc_tiling_on_sc=True, needs_layout_passes=False),
    )
    def k(idx_hbm, table_hbm, out_hbm, *refs):
        idx_bufs = refs[0:2]
        rows_bufs = refs[2:4]
        tr_bufs = refs[4:6]
        skew_buf = refs[6]
        sem_i = refs[7:9]
        sem_g = refs[9:11]
        sem_o = refs[11:13]

        nc = 2
        wid = lax.axis_index("s") * nc + lax.axis_index("c")
        base = wid * b_per_w

        def issue_idx(g, b):
            pltpu.async_copy(
                idx_hbm.at[pl.ds(base + g * CHUNK, CHUNK)], idx_bufs[b], sem_i[b]
            )

        def wait_idx(b):
            pltpu.make_async_copy(
                idx_hbm.at[pl.ds(0, CHUNK)], idx_bufs[b], sem_i[b]
            ).wait()

        def issue_gather(b):
            pltpu.async_copy(table_hbm.at[idx_bufs[b]], rows_bufs[b], sem_g[b])

        def wait_gather(b):
            pltpu.make_async_copy(
                table_hbm.at[idx_bufs[b]], rows_bufs[b], sem_g[b]
            ).wait()

        def issue_out(g, b):
            pltpu.async_copy(
                tr_bufs[b],
                out_hbm.at[:, pl.ds(base + g * CHUNK, CHUNK)],
                sem_o[b],
            )

        def wait_out(b):
            pltpu.make_async_copy(
                tr_bufs[b], out_hbm.at[:, pl.ds(0, CHUNK)], sem_o[b]
            ).wait()

        lanes = lax.iota(jnp.int32, 16)
        rot = [
            jnp.where(lanes + r < 16, lanes + r, lanes + r - 16)
            for r in range(16)
        ]
        lanes64 = lanes * 64

        def transpose_chunk(rows_buf, tr_buf):
            @plsc.parallel_loop(0, CHUNK // 16, 1, unroll=2)
            def skew_pass(cb):
                c0 = pl.multiple_of(cb * 16, 16)
                sbase = c0 * 64
                for r in range(16):
                    for dg in range(4):
                        v = rows_buf[c0 + r, pl.ds(dg * 16, 16)]
                        plsc.store_scatter(
                            skew_buf, [sbase + r * 64 + dg * 16 + rot[r]], v
                        )

            @plsc.parallel_loop(0, CHUNK // 16, 1, unroll=2)
            def out_pass(cb):
                c0 = pl.multiple_of(cb * 16, 16)
                sbase = c0 * 64
                for dg in range(4):
                    for k in range(16):
                        iv = sbase + lanes64 + (dg * 16 + rot[k])
                        v = plsc.load_gather(skew_buf, [iv])
                        tr_buf[dg * 16 + k, pl.ds(c0, 16)] = v

        issue_idx(0, 0)
        wait_idx(0)
        issue_gather(0)
        issue_idx(1, 1)

        def body(g, carry):
            b = lax.rem(g, 2)
            @pl.when(g + 1 < n_chunks)
            def _():
                for bb in range(2):
                    @pl.when(b != bb)
                    def _():
                        wait_idx(bb)
                        issue_gather(bb)

            @pl.when(g + 2 < n_chunks)
            def _():
                for bb in range(2):
                    @pl.when(b == bb)
                    def _():
                        wait_gather(bb)
                        issue_idx(g + 2, bb)

            for bb in range(2):
                @pl.when(b == bb)
                def _():
                    @pl.when(g + 2 >= n_chunks)
                    def _():
                        wait_gather(bb)
                    @pl.when(g >= 2)
                    def _():
                        wait_out(bb)
                    transpose_chunk(rows_bufs[bb], tr_bufs[bb])
                    issue_out(g, bb)
            return carry

        lax.fori_loop(0, n_chunks, body, 0)

        for bb in range(2):
            wait_out(bb)

    return k


def kernel(patch_indices, patch_embeddings):
    B = patch_indices.shape[0]
    V, D = patch_embeddings.shape
    idx = patch_indices.astype(jnp.int32)
    table_pad = jnp.pad(patch_embeddings, ((0, 0), (0, DPAD - D)))
    out_t = _gather_kernel(B, V, D)(idx, table_pad)
    return out_t.T


# speedup vs baseline: 1.0562x; 1.0031x over previous
import functools

import jax
import jax.numpy as jnp
from jax import lax
from jax.experimental import pallas as pl
from jax.experimental.pallas import tpu as pltpu
from jax.experimental.pallas import tpu_sc as plsc

NW = 32
CHUNK = 256
DPAD = 128


def _gather_kernel(B, V, D):
    b_per_w = B // NW
    n_chunks = b_per_w // CHUNK
    mesh = plsc.VectorSubcoreMesh(core_axis_name="c", subcore_axis_name="s")

    scratch = (
        [pltpu.VMEM((CHUNK,), jnp.int32) for _ in range(2)]
        + [pltpu.VMEM((CHUNK, DPAD), jnp.float32) for _ in range(2)]
        + [pltpu.VMEM((D, CHUNK), jnp.float32) for _ in range(2)]
        + [pltpu.VMEM((CHUNK * 64,), jnp.float32)]
        + [pltpu.SemaphoreType.DMA for _ in range(6)]
    )

    @functools.partial(
        pl.kernel,
        out_type=jax.ShapeDtypeStruct((D, B), jnp.float32),
        mesh=mesh,
        scratch_types=scratch,
        compiler_params=pltpu.CompilerParams(use_tc_tiling_on_sc=True, needs_layout_passes=False),
    )
    def k(idx_hbm, table_hbm, out_hbm, *refs):
        idx_bufs = refs[0:2]
        rows_bufs = refs[2:4]
        tr_bufs = refs[4:6]
        skew_buf = refs[6]
        sem_i = refs[7:9]
        sem_g = refs[9:11]
        sem_o = refs[11:13]

        nc = 2
        wid = lax.axis_index("s") * nc + lax.axis_index("c")
        base = wid * b_per_w

        def issue_idx(g, b):
            pltpu.async_copy(
                idx_hbm.at[pl.ds(base + g * CHUNK, CHUNK)], idx_bufs[b], sem_i[b]
            )

        def wait_idx(b):
            pltpu.make_async_copy(
                idx_hbm.at[pl.ds(0, CHUNK)], idx_bufs[b], sem_i[b]
            ).wait()

        def issue_gather(b):
            pltpu.async_copy(table_hbm.at[idx_bufs[b]], rows_bufs[b], sem_g[b])

        def wait_gather(b):
            pltpu.make_async_copy(
                table_hbm.at[idx_bufs[b]], rows_bufs[b], sem_g[b]
            ).wait()

        def issue_out(g, b):
            pltpu.async_copy(
                tr_bufs[b],
                out_hbm.at[:, pl.ds(base + g * CHUNK, CHUNK)],
                sem_o[b],
            )

        def wait_out(b):
            pltpu.make_async_copy(
                tr_bufs[b], out_hbm.at[:, pl.ds(0, CHUNK)], sem_o[b]
            ).wait()

        lanes = lax.iota(jnp.int32, 16)
        drows = [lanes + dg * 16 for dg in range(4)]

        def transpose_chunk(rows_buf, tr_buf):
            @plsc.parallel_loop(0, CHUNK, 1, unroll=8)
            def cbody(c):
                colv = jnp.full((16,), c, dtype=jnp.int32)
                for dg in range(4):
                    v = rows_buf[c, pl.ds(dg * 16, 16)]
                    plsc.store_scatter(tr_buf, [drows[dg], colv], v)

        issue_idx(0, 0)
        wait_idx(0)
        issue_gather(0)
        issue_idx(1, 1)

        def body(g, carry):
            b = lax.rem(g, 2)
            @pl.when(g + 1 < n_chunks)
            def _():
                for bb in range(2):
                    @pl.when(b != bb)
                    def _():
                        wait_idx(bb)
                        issue_gather(bb)

            @pl.when(g + 2 < n_chunks)
            def _():
                for bb in range(2):
                    @pl.when(b == bb)
                    def _():
                        wait_gather(bb)
                        issue_idx(g + 2, bb)

            for bb in range(2):
                @pl.when(b == bb)
                def _():
                    @pl.when(g + 2 >= n_chunks)
                    def _():
                        wait_gather(bb)
                    @pl.when(g >= 2)
                    def _():
                        wait_out(bb)
                    transpose_chunk(rows_bufs[bb], tr_bufs[bb])
                    issue_out(g, bb)
            return carry

        lax.fori_loop(0, n_chunks, body, 0)

        for bb in range(2):
            wait_out(bb)

    return k


def kernel(patch_indices, patch_embeddings):
    B = patch_indices.shape[0]
    V, D = patch_embeddings.shape
    idx = patch_indices.astype(jnp.int32)
    table_pad = jnp.pad(patch_embeddings, ((0, 0), (0, DPAD - D)))
    out_t = _gather_kernel(B, V, D)(idx, table_pad)
    return out_t.T
